# baseline (speedup 1.0000x reference)
import functools

import jax
import jax.numpy as jnp
from jax import lax
from jax.experimental import pallas as pl
from jax.experimental.pallas import tpu as pltpu
from jax.experimental.pallas import tpu_sc as plsc

B = 16384
V = 100000
E = 64
H = 128
A = 8

_info = plsc.get_sparse_core_info()
NC = _info.num_cores
NS = _info.num_subcores
L = _info.num_lanes
NW = NC * NS
BPW = B // NW
CH = 128
NCH = BPW // CH
Q = E // 2

TBLK = 4096
NTBLK = 7
SPLIT = NTBLK * TBLK
VPACK = SPLIT
LASTBLK = (V - 1) // TBLK

_mesh = plsc.VectorSubcoreMesh(core_axis_name="c", subcore_axis_name="s")


def _transpose_body(x0, x1, x2, x3, o_ref):
    parts = []
    for x in (x0, x1, x2, x3):
        xT = lax.transpose(x[...], (1, 0))
        xr = xT.astype(jnp.bfloat16).astype(jnp.float32)
        bits = lax.bitcast_convert_type(xr, jnp.int32)
        a = bits[:, 0:Q]
        b = bits[:, Q:E]
        w = lax.bitwise_or(lax.shift_right_logical(a, 16),
                           lax.bitwise_and(b, jnp.int32(-65536)))
        parts.append(w)
    o_ref[...] = jnp.concatenate(parts, axis=1)


def _transpose_pack(table_t):
    return pl.pallas_call(
        _transpose_body,
        grid=(NTBLK,),
        in_specs=[
            pl.BlockSpec((E, TBLK),
                         lambda i, k=k: (0, jnp.minimum(i + k * NTBLK, LASTBLK)))
            for k in range(4)
        ],
        out_specs=pl.BlockSpec((TBLK, 2 * E), lambda i: (i, 0)),
        out_shape=jax.ShapeDtypeStruct((VPACK, 2 * E), jnp.int32),
    )(table_t, table_t, table_t, table_t)


def _make_gather(ks, with_partial=False):
    nk = len(ks)

    @functools.partial(
        pl.kernel,
        out_type=jax.ShapeDtypeStruct((B, E), jnp.float32),
        mesh=_mesh,
        scratch_types=[
            pltpu.VMEM((4, BPW), jnp.int32),
            pltpu.VMEM((4, BPW), jnp.int32),
            pltpu.VMEM((4, CH, 2 * E), jnp.int32),
            pltpu.VMEM((CH, E), jnp.float32),
            pltpu.VMEM((CH, E), jnp.float32),
            pltpu.SemaphoreType.DMA,
        ],
        compiler_params=pltpu.CompilerParams(needs_layout_passes=False),
    )
    def _gather_sum(idxr_hbm, oq_hbm, *refs):
        tables = refs[:nk]
        if with_partial:
            part_hbm = refs[nk]
            out_hbm, idxr_v, oq_v, buf, pbuf, acc, sem = refs[nk + 1:]
        else:
            part_hbm = None
            out_hbm, idxr_v, oq_v, buf, pbuf, acc, sem = refs[nk:]
        wid = lax.axis_index("s") * NC + lax.axis_index("c")
        base = wid * BPW
        for j, k in enumerate(ks):
            pltpu.sync_copy(idxr_hbm.at[k, pl.ds(base, BPW)], idxr_v.at[j])
            pltpu.sync_copy(oq_hbm.at[k, pl.ds(base, BPW)], oq_v.at[j])

        for ch in range(NCH):
            cps = [
                pltpu.async_copy(
                    tables[j].at[idxr_v.at[j, pl.ds(ch * CH, CH)]],
                    buf.at[j],
                    sem,
                )
                for j in range(nk)
            ]
            if with_partial:
                cps.append(pltpu.async_copy(
                    part_hbm.at[pl.ds(base + ch * CH, CH)], pbuf, sem))
            for cp in cps:
                cp.wait()

            def _grp(g, carry):
                rows = lax.iota(jnp.int32, L) + g * L
                ovecs = [oq_v[j, pl.ds(ch * CH + g * L, L)] for j in range(nk)]
                lane = lax.iota(jnp.int32, L)
                for c in range(Q):
                    colpat = lax.rem(lane + c, Q)
                    if with_partial:
                        s_lo = plsc.load_gather(pbuf, [rows, colpat])
                        s_hi = plsc.load_gather(pbuf, [rows, colpat + Q])
                    else:
                        s_lo = None
                        s_hi = None
                    for j in range(nk):
                        ww = plsc.load_gather(buf.at[j],
                                              [rows, ovecs[j] + colpat])
                        lo = plsc.bitcast(lax.shift_left(ww, 16), jnp.float32)
                        hi = plsc.bitcast(
                            lax.bitwise_and(ww, jnp.int32(-65536)), jnp.float32)
                        s_lo = lo if s_lo is None else s_lo + lo
                        s_hi = hi if s_hi is None else s_hi + hi
                    plsc.store_scatter(acc, [rows, colpat], s_lo)
                    plsc.store_scatter(acc, [rows, colpat + Q], s_hi)
                return carry

            lax.fori_loop(0, CH // L, _grp, 0)
            pltpu.sync_copy(acc, out_hbm.at[pl.ds(base + ch * CH, CH)])

    return _gather_sum


_gather3 = _make_gather((0, 1, 2))
_gather1 = _make_gather((3,), with_partial=True)


def _mlp_body(x_ref, w1_ref, b1_ref, w2_ref, b2_ref, wa_ref, ba_ref, o_ref):
    x = x_ref[...]
    h = jnp.dot(x, w1_ref[...], preferred_element_type=jnp.float32) + b1_ref[...]
    h = jnp.maximum(h, 0.0)
    h = jnp.dot(h, w2_ref[...], preferred_element_type=jnp.float32) + b2_ref[...]
    h = jnp.maximum(h, 0.0)
    o_ref[...] = lax.dot_general(wa_ref[...], h, (((0,), (1,)), ((), ())),
                                 preferred_element_type=jnp.float32) + ba_ref[...]


def _mlp(x, w1, b1, w2, b2, wa, ba):
    BT = 4096
    return pl.pallas_call(
        _mlp_body,
        grid=(B // BT,),
        in_specs=[
            pl.BlockSpec((BT, E), lambda i: (i, 0)),
            pl.BlockSpec((E, H), lambda i: (0, 0)),
            pl.BlockSpec((1, H), lambda i: (0, 0)),
            pl.BlockSpec((H, H), lambda i: (0, 0)),
            pl.BlockSpec((1, H), lambda i: (0, 0)),
            pl.BlockSpec((H, A), lambda i: (0, 0)),
            pl.BlockSpec((A, 1), lambda i: (0, 0)),
        ],
        out_specs=pl.BlockSpec((A, BT), lambda i: (0, i)),
        out_shape=jax.ShapeDtypeStruct((A, B), jnp.float32),
    )(x, w1, b1.reshape(1, H), w2, b2.reshape(1, H), wa, ba.reshape(A, 1))


def kernel(inputs, emb_fid, emb_lba, emb_bytes, emb_bblba, w1, b1, w2, b2, wa, ba):
    idx_t = inputs.astype(jnp.int32).T
    idxr_t = idx_t % SPLIT
    oq_t = (idx_t // SPLIT) * Q
    tp = [_transpose_pack(t.T) for t in (emb_fid, emb_lba, emb_bytes, emb_bblba)]
    part_a = _gather3(idxr_t, oq_t, tp[0], tp[1], tp[2])
    summed = _gather1(idxr_t, oq_t, tp[3], part_a)
    out_t = _mlp(summed, w1, b1, w2, b2, wa, ba)
    return out_t.T


# speedup vs baseline: 1.0080x; 1.0080x over previous
import functools

import jax
import jax.numpy as jnp
from jax import lax
from jax.experimental import pallas as pl
from jax.experimental.pallas import tpu as pltpu
from jax.experimental.pallas import tpu_sc as plsc

B = 16384
V = 100000
E = 64
H = 128
A = 8

_info = plsc.get_sparse_core_info()
NC = _info.num_cores
NS = _info.num_subcores
L = _info.num_lanes
NW = NC * NS
BPW = B // NW
CH = 128
NCH = BPW // CH
Q = E // 2

TBLK = 4096
NTBLK = 7
SPLIT = NTBLK * TBLK
VPACK = SPLIT
LASTBLK = (V - 1) // TBLK

_mesh = plsc.VectorSubcoreMesh(core_axis_name="c", subcore_axis_name="s")


def _transpose_body(x0, x1, x2, x3, o_ref):
    parts = []
    for x in (x0, x1, x2, x3):
        xT = lax.transpose(x[...], (1, 0))
        xr = xT.astype(jnp.bfloat16).astype(jnp.float32)
        bits = lax.bitcast_convert_type(xr, jnp.int32)
        a = bits[:, 0:Q]
        b = bits[:, Q:E]
        w = lax.bitwise_or(lax.shift_right_logical(a, 16),
                           lax.bitwise_and(b, jnp.int32(-65536)))
        parts.append(w)
    o_ref[...] = jnp.concatenate(parts, axis=1)


def _transpose_pack(table_t):
    return pl.pallas_call(
        _transpose_body,
        grid=(NTBLK,),
        in_specs=[
            pl.BlockSpec((E, TBLK),
                         lambda i, k=k: (0, jnp.minimum(i + k * NTBLK, LASTBLK)))
            for k in range(4)
        ],
        out_specs=pl.BlockSpec((TBLK, 2 * E), lambda i: (i, 0)),
        out_shape=jax.ShapeDtypeStruct((VPACK, 2 * E), jnp.int32),
    )(table_t, table_t, table_t, table_t)


def _make_gather(ks):
    nk = len(ks)

    @functools.partial(
        pl.kernel,
        out_type=jax.ShapeDtypeStruct((B, E), jnp.float32),
        mesh=_mesh,
        scratch_types=[
            pltpu.VMEM((4, BPW), jnp.int32),
            pltpu.VMEM((4, BPW), jnp.int32),
            pltpu.VMEM((4, CH, 2 * E), jnp.int32),
            pltpu.VMEM((CH, E), jnp.float32),
            pltpu.SemaphoreType.DMA,
        ],
        compiler_params=pltpu.CompilerParams(needs_layout_passes=False),
    )
    def _gather_sum(idxr_hbm, oq_hbm, *refs):
        tables = refs[:nk]
        out_hbm, idxr_v, oq_v, buf, acc, sem = refs[nk:]
        wid = lax.axis_index("s") * NC + lax.axis_index("c")
        base = wid * BPW
        for j, k in enumerate(ks):
            pltpu.sync_copy(idxr_hbm.at[k, pl.ds(base, BPW)], idxr_v.at[j])
            pltpu.sync_copy(oq_hbm.at[k, pl.ds(base, BPW)], oq_v.at[j])

        for ch in range(NCH):
            cps = [
                pltpu.async_copy(
                    tables[j].at[idxr_v.at[j, pl.ds(ch * CH, CH)]],
                    buf.at[j],
                    sem,
                )
                for j in range(nk)
            ]
            for cp in cps:
                cp.wait()

            def _grp(g, carry):
                rows = lax.iota(jnp.int32, L) + g * L
                ovecs = [oq_v[j, pl.ds(ch * CH + g * L, L)] for j in range(nk)]
                lane = lax.iota(jnp.int32, L)
                for c in range(Q):
                    colpat = lax.rem(lane + c, Q)
                    s_lo = None
                    s_hi = None
                    for j in range(nk):
                        ww = plsc.load_gather(buf.at[j],
                                              [rows, ovecs[j] + colpat])
                        lo = plsc.bitcast(lax.shift_left(ww, 16), jnp.float32)
                        hi = plsc.bitcast(
                            lax.bitwise_and(ww, jnp.int32(-65536)), jnp.float32)
                        s_lo = lo if s_lo is None else s_lo + lo
                        s_hi = hi if s_hi is None else s_hi + hi
                    plsc.store_scatter(acc, [rows, colpat], s_lo)
                    plsc.store_scatter(acc, [rows, colpat + Q], s_hi)
                return carry

            lax.fori_loop(0, CH // L, _grp, 0)
            pltpu.sync_copy(acc, out_hbm.at[pl.ds(base + ch * CH, CH)])

    return _gather_sum


_gather3 = _make_gather((0, 1, 2))
_gather1 = _make_gather((3,))


def _mlp_body(xa_ref, xb_ref, w1_ref, b1_ref, w2_ref, b2_ref, wa_ref, ba_ref, o_ref):
    x = xa_ref[...] + xb_ref[...]
    h = jnp.dot(x, w1_ref[...], preferred_element_type=jnp.float32) + b1_ref[...]
    h = jnp.maximum(h, 0.0)
    h = jnp.dot(h, w2_ref[...], preferred_element_type=jnp.float32) + b2_ref[...]
    h = jnp.maximum(h, 0.0)
    o_ref[...] = lax.dot_general(wa_ref[...], h, (((0,), (1,)), ((), ())),
                                 preferred_element_type=jnp.float32) + ba_ref[...]


def _mlp(xa, xb, w1, b1, w2, b2, wa, ba):
    BT = 2048
    return pl.pallas_call(
        _mlp_body,
        grid=(B // BT,),
        in_specs=[
            pl.BlockSpec((BT, E), lambda i: (i, 0)),
            pl.BlockSpec((BT, E), lambda i: (i, 0)),
            pl.BlockSpec((E, H), lambda i: (0, 0)),
            pl.BlockSpec((1, H), lambda i: (0, 0)),
            pl.BlockSpec((H, H), lambda i: (0, 0)),
            pl.BlockSpec((1, H), lambda i: (0, 0)),
            pl.BlockSpec((H, A), lambda i: (0, 0)),
            pl.BlockSpec((A, 1), lambda i: (0, 0)),
        ],
        out_specs=pl.BlockSpec((A, BT), lambda i: (0, i)),
        out_shape=jax.ShapeDtypeStruct((A, B), jnp.float32),
    )(xa, xb, w1, b1.reshape(1, H), w2, b2.reshape(1, H), wa, ba.reshape(A, 1))


def kernel(inputs, emb_fid, emb_lba, emb_bytes, emb_bblba, w1, b1, w2, b2, wa, ba):
    idx_t = inputs.astype(jnp.int32).T
    idxr_t = idx_t % SPLIT
    oq_t = (idx_t // SPLIT) * Q
    tp = [_transpose_pack(t.T) for t in (emb_fid, emb_lba, emb_bytes, emb_bblba)]
    part_a = _gather3(idxr_t, oq_t, tp[0], tp[1], tp[2])
    part_b = _gather1(idxr_t, oq_t, tp[3])
    out_t = _mlp(part_a, part_b, w1, b1, w2, b2, wa, ba)
    return out_t.T
